# Initial kernel scaffold; baseline (speedup 1.0000x reference)
import jax
import jax.numpy as jnp
from jax.experimental import pallas as pl


def kernel(x, edge_index, W1, b1, gamma1, beta1, W2, b2, gamma2, beta2):
    raise NotImplementedError("write your pallas kernel here")


# speedup vs baseline: 9.6232x; 9.6232x over previous
import functools

import jax
import jax.numpy as jnp
from jax import lax
from jax.experimental import pallas as pl
from jax.experimental.pallas import tpu as pltpu
from jax.experimental.pallas import tpu_sc as plsc

N = 10000
HALF = 128
NT = 16
CHUNK = 128
NCH = 80
EPT = NCH * CHUNK
EPAD = NT * EPT
ACC_ROWS = N + 16
SLAB = 632


def _slab_base(s, total):
    b = jnp.where(s < NT - 1, s * SLAB, total - SLAB)
    return pl.multiple_of(b, 8)

@functools.cache
def _mesh():
    return plsc.VectorSubcoreMesh(core_axis_name="c", subcore_axis_name="s")


def _sc_scatter_body(hpA, hpB, src3, dst3, outA, outB, sidx, didx, rows, acc,
                     gsem):
    c = lax.axis_index("c")
    s = lax.axis_index("s")

    def run(hp, out):
        base = _slab_base(s, N)
        pltpu.sync_copy(hp.at[pl.ds(base, SLAB)], acc.at[pl.ds(base, SLAB)])
        pltpu.sync_copy(src3.at[s], sidx)
        pltpu.sync_copy(dst3.at[s], didx)
        plsc.subcore_barrier()

        def step(j, carry):
            pltpu.async_copy(hp.at[sidx.at[j]], rows, gsem).wait()
            pltpu.sync_copy(rows, acc.at[didx.at[j]], add=True)
            return carry

        lax.fori_loop(0, NCH, step, 0)
        plsc.subcore_barrier()
        pltpu.sync_copy(acc.at[pl.ds(base, SLAB)], out.at[pl.ds(base, SLAB)])

    @pl.when(c == 0)
    def _():
        run(hpA, outA)

    @pl.when(c == 1)
    def _():
        run(hpB, outB)


@functools.cache
def _sc_scatter():
    return pl.kernel(
        _sc_scatter_body,
        out_type=(jax.ShapeDtypeStruct((N, HALF), jnp.float32),
                  jax.ShapeDtypeStruct((N, HALF), jnp.float32)),
        mesh=_mesh(),
        scratch_types=[
            pltpu.VMEM((NCH, CHUNK), jnp.int32),
            pltpu.VMEM((NCH, CHUNK), jnp.int32),
            pltpu.VMEM((CHUNK, HALF), jnp.float32),
            pltpu.VMEM_SHARED((ACC_ROWS, HALF), jnp.float32),
            pltpu.SemaphoreType.DMA,
        ],
    )


def _mm_body(x_ref, w_ref, deg_ref, oa_ref, ob_ref):
    xs = x_ref[...] * lax.rsqrt(deg_ref[...])
    h = jnp.dot(xs, w_ref[...], preferred_element_type=jnp.float32,
                precision=lax.Precision.HIGHEST)
    oa_ref[...] = h[:, :HALF]
    ob_ref[...] = h[:, HALF:]


def _mm(x, w, deg2d):
    m, k = x.shape
    bm = 1000
    return pl.pallas_call(
        _mm_body,
        grid=(m // bm,),
        in_specs=[
            pl.BlockSpec((bm, k), lambda i: (i, 0)),
            pl.BlockSpec((k, 2 * HALF), lambda i: (0, 0)),
            pl.BlockSpec((bm, 1), lambda i: (i, 0)),
        ],
        out_specs=[
            pl.BlockSpec((bm, HALF), lambda i: (i, 0)),
            pl.BlockSpec((bm, HALF), lambda i: (i, 0)),
        ],
        out_shape=[jax.ShapeDtypeStruct((m, HALF), jnp.float32)] * 2,
    )(x, w, deg2d)


def _bn_body(a_ref, b_ref, deg_ref, bias_ref, g_ref, bt_ref, o_ref):
    dv = lax.rsqrt(deg_ref[...])
    for half, ref in ((0, a_ref), (1, b_ref)):
        sl = pl.ds(half * HALF, HALF)
        z = ref[...] * dv + bias_ref[:, sl]
        mu = jnp.mean(z, axis=0, keepdims=True)
        zc = z - mu
        var = jnp.mean(zc * zc, axis=0, keepdims=True)
        y = zc * lax.rsqrt(var + 1e-5) * g_ref[:, sl] + bt_ref[:, sl]
        o_ref[:, sl] = jnp.maximum(y, 0.0)


def _bn(acc_a, acc_b, deg2d, bias, gamma, beta):
    return pl.pallas_call(
        _bn_body,
        out_shape=jax.ShapeDtypeStruct((N, 2 * HALF), jnp.float32),
    )(acc_a, acc_b, deg2d, bias.reshape(1, -1), gamma.reshape(1, -1),
      beta.reshape(1, -1))


def kernel(x, edge_index, W1, b1, gamma1, beta1, W2, b2, gamma2, beta2):
    src = edge_index[0]
    dst = edge_index[1]
    e = src.shape[0]
    npad = EPAD - e
    ar = jnp.arange(npad, dtype=jnp.int32)
    psrc = (ar * 613) % N
    pdst = N + (ar % 16)
    src3 = jnp.concatenate([src, psrc]).reshape(NT, NCH, CHUNK)
    dst3 = jnp.concatenate([dst, pdst]).reshape(NT, NCH, CHUNK)

    ones_n = jnp.ones((N, HALF), jnp.float32)
    deg_a, _ = _sc_scatter()(ones_n, ones_n, src3, dst3)
    deg2d = deg_a[:, :1]

    hp_a, hp_b = _mm(x, W1, deg2d)
    acc_a, acc_b = _sc_scatter()(hp_a, hp_b, src3, dst3)
    y1 = _bn(acc_a, acc_b, deg2d, b1, gamma1, beta1)

    hp_a, hp_b = _mm(y1, W2, deg2d)
    acc_a, acc_b = _sc_scatter()(hp_a, hp_b, src3, dst3)
    return _bn(acc_a, acc_b, deg2d, b2, gamma2, beta2)


# speedup vs baseline: 16.4288x; 1.7072x over previous
import functools

import jax
import jax.numpy as jnp
from jax import lax
from jax.experimental import pallas as pl
from jax.experimental.pallas import tpu as pltpu
from jax.experimental.pallas import tpu_sc as plsc

N = 10000
HALF = 128
NT = 16
CHUNK = 128
NCH = 80
NRND = 2
NCHR = NCH // NRND
EPT = NCH * CHUNK
EPAD = NT * EPT
ACC_ROWS = N + 16
SLAB = 632


def _slab_base(s, total):
    b = jnp.where(s < NT - 1, s * SLAB, total - SLAB)
    return pl.multiple_of(b, 8)

@functools.cache
def _mesh():
    return plsc.VectorSubcoreMesh(core_axis_name="c", subcore_axis_name="s")


def _sc_scatter_body(hpA, hpB, src4, dst4, outA, outB, sidx, didx, rows0,
                     rows1, acc, gsem0, gsem1, ssem0, ssem1):
    c = lax.axis_index("c")
    s = lax.axis_index("s")
    rows = (rows0, rows1)
    gsem = (gsem0, gsem1)
    ssem = (ssem0, ssem1)

    def run(hp, out):
        base = _slab_base(s, N)
        pltpu.sync_copy(hp.at[pl.ds(base, SLAB)], acc.at[pl.ds(base, SLAB)])

        def g_start(j, b):
            pltpu.async_copy(hp.at[sidx.at[j]], rows[b], gsem[b])

        def g_wait(j, b):
            pltpu.make_async_copy(hp.at[sidx.at[j]], rows[b], gsem[b]).wait()

        def s_start(j, b):
            pltpu.async_copy(rows[b], acc.at[didx.at[j]], ssem[b], add=True)

        def s_wait(j, b):
            pltpu.make_async_copy(rows[b], acc.at[didx.at[j]], ssem[b]).wait()

        for r in range(NRND):
            pltpu.sync_copy(src4.at[s, r], sidx)
            pltpu.sync_copy(dst4.at[s, r], didx)
            if r == 0:
                plsc.subcore_barrier()

            g_start(0, 0)

            @pl.loop(0, NCHR, step=2)
            def _(jj):
                for b in range(2):
                    j = jj + b

                    @pl.when(j > 0)
                    def _():
                        s_wait(j - 1, 1 - b)

                    @pl.when(j + 1 < NCHR)
                    def _():
                        g_start(j + 1, 1 - b)

                    g_wait(j, b)
                    s_start(j, b)

            s_wait(NCHR - 1, 1)

        plsc.subcore_barrier()
        pltpu.sync_copy(acc.at[pl.ds(base, SLAB)], out.at[pl.ds(base, SLAB)])

    @pl.when(c == 0)
    def _():
        run(hpA, outA)

    @pl.when(c == 1)
    def _():
        run(hpB, outB)


@functools.cache
def _sc_scatter():
    return pl.kernel(
        _sc_scatter_body,
        out_type=(jax.ShapeDtypeStruct((N, HALF), jnp.float32),
                  jax.ShapeDtypeStruct((N, HALF), jnp.float32)),
        mesh=_mesh(),
        scratch_types=[
            pltpu.VMEM((NCHR, CHUNK), jnp.int32),
            pltpu.VMEM((NCHR, CHUNK), jnp.int32),
            pltpu.VMEM((CHUNK, HALF), jnp.float32),
            pltpu.VMEM((CHUNK, HALF), jnp.float32),
            pltpu.VMEM_SHARED((ACC_ROWS, HALF), jnp.float32),
            pltpu.SemaphoreType.DMA,
            pltpu.SemaphoreType.DMA,
            pltpu.SemaphoreType.DMA,
            pltpu.SemaphoreType.DMA,
        ],
    )


def _sc_deg_body(dst4, ones_hbm, outA, outB, didx, ones_v, acc, ssem):
    c = lax.axis_index("c")
    s = lax.axis_index("s")
    base = _slab_base(s, N)

    pltpu.sync_copy(ones_hbm, acc.at[pl.ds(base, SLAB)])
    pltpu.sync_copy(ones_hbm.at[pl.ds(0, CHUNK)], ones_v)
    pltpu.sync_copy(dst4.at[s, c], didx)
    plsc.subcore_barrier()

    @pl.loop(0, NCHR, step=8)
    def _(t0):
        for t in range(8):
            pltpu.async_copy(ones_v, acc.at[didx.at[t0 + t]], ssem, add=True)
        for t in range(8):
            pltpu.make_async_copy(ones_v, acc.at[didx.at[t0 + t]],
                                  ssem).wait()

    plsc.subcore_barrier()

    @pl.when(c == 0)
    def _():
        pltpu.sync_copy(acc.at[pl.ds(base, SLAB)], outA.at[pl.ds(base, SLAB)])

    @pl.when(c == 1)
    def _():
        pltpu.sync_copy(acc.at[pl.ds(base, SLAB)], outB.at[pl.ds(base, SLAB)])


@functools.cache
def _sc_deg():
    return pl.kernel(
        _sc_deg_body,
        out_type=(jax.ShapeDtypeStruct((N, HALF), jnp.float32),
                  jax.ShapeDtypeStruct((N, HALF), jnp.float32)),
        mesh=_mesh(),
        scratch_types=[
            pltpu.VMEM((NCHR, CHUNK), jnp.int32),
            pltpu.VMEM((CHUNK, HALF), jnp.float32),
            pltpu.VMEM_SHARED((ACC_ROWS, HALF), jnp.float32),
            pltpu.SemaphoreType.DMA,
        ],
    )


def _mm_body(x_ref, w_ref, da_ref, db_ref, oa_ref, ob_ref):
    xs = x_ref[...] * lax.rsqrt(da_ref[:, :1] + db_ref[:, :1] - 1.0)
    h = jnp.dot(xs, w_ref[...], preferred_element_type=jnp.float32,
                precision=lax.Precision.HIGHEST)
    oa_ref[...] = h[:, :HALF]
    ob_ref[...] = h[:, HALF:]


def _mm(x, w, deg_a, deg_b):
    m, k = x.shape
    bm = 1000
    return pl.pallas_call(
        _mm_body,
        grid=(m // bm,),
        in_specs=[
            pl.BlockSpec((bm, k), lambda i: (i, 0)),
            pl.BlockSpec((k, 2 * HALF), lambda i: (0, 0)),
            pl.BlockSpec((bm, HALF), lambda i: (i, 0)),
            pl.BlockSpec((bm, HALF), lambda i: (i, 0)),
        ],
        out_specs=[
            pl.BlockSpec((bm, HALF), lambda i: (i, 0)),
            pl.BlockSpec((bm, HALF), lambda i: (i, 0)),
        ],
        out_shape=[jax.ShapeDtypeStruct((m, HALF), jnp.float32)] * 2,
    )(x, w, deg_a, deg_b)


def _bn_body(a_ref, b_ref, da_ref, db_ref, bias_ref, g_ref, bt_ref, o_ref):
    dv = lax.rsqrt(da_ref[:, :1] + db_ref[:, :1] - 1.0)
    for half, ref in ((0, a_ref), (1, b_ref)):
        sl = pl.ds(half * HALF, HALF)
        z = ref[...] * dv + bias_ref[:, sl]
        mu = jnp.mean(z, axis=0, keepdims=True)
        zc = z - mu
        var = jnp.mean(zc * zc, axis=0, keepdims=True)
        y = zc * lax.rsqrt(var + 1e-5) * g_ref[:, sl] + bt_ref[:, sl]
        o_ref[:, sl] = jnp.maximum(y, 0.0)


def _bn(acc_a, acc_b, deg_a, deg_b, bias, gamma, beta):
    return pl.pallas_call(
        _bn_body,
        in_specs=[
            pl.BlockSpec((N, HALF), lambda: (0, 0)),
            pl.BlockSpec((N, HALF), lambda: (0, 0)),
            pl.BlockSpec((N, HALF), lambda: (0, 0)),
            pl.BlockSpec((N, HALF), lambda: (0, 0)),
            pl.BlockSpec((1, 2 * HALF), lambda: (0, 0)),
            pl.BlockSpec((1, 2 * HALF), lambda: (0, 0)),
            pl.BlockSpec((1, 2 * HALF), lambda: (0, 0)),
        ],
        out_shape=jax.ShapeDtypeStruct((N, 2 * HALF), jnp.float32),
    )(acc_a, acc_b, deg_a, deg_b, bias.reshape(1, -1), gamma.reshape(1, -1),
      beta.reshape(1, -1))


def kernel(x, edge_index, W1, b1, gamma1, beta1, W2, b2, gamma2, beta2):
    src = edge_index[0]
    dst = edge_index[1]
    e = src.shape[0]
    npad = EPAD - e
    ar = jnp.arange(npad, dtype=jnp.int32)
    psrc = (ar * 613) % N
    pdst = N + (ar % 16)
    src4 = jnp.concatenate([src, psrc]).reshape(NT, NRND, NCHR, CHUNK)
    dst4 = jnp.concatenate([dst, pdst]).reshape(NT, NRND, NCHR, CHUNK)

    ones_slab = jnp.ones((SLAB, HALF), jnp.float32)
    deg_a, deg_b = _sc_deg()(dst4, ones_slab)

    hp_a, hp_b = _mm(x, W1, deg_a, deg_b)
    acc_a, acc_b = _sc_scatter()(hp_a, hp_b, src4, dst4)
    y1 = _bn(acc_a, acc_b, deg_a, deg_b, b1, gamma1, beta1)

    hp_a, hp_b = _mm(y1, W2, deg_a, deg_b)
    acc_a, acc_b = _sc_scatter()(hp_a, hp_b, src4, dst4)
    return _bn(acc_a, acc_b, deg_a, deg_b, b2, gamma2, beta2)
